# baseline (speedup 1.0000x reference)
import functools

import jax
import jax.numpy as jnp
from jax import lax
from jax.experimental import pallas as pl
from jax.experimental.pallas import tpu as pltpu
from jax.experimental.pallas import tpu_sc as plsc

N = 10000
D = 128
HD = D // 2
E = 320000
NPAD = 10240
NC, NS = 2, 16
NW = NC * NS
CH = 128
ECH = E // CH
NCH = ECH // NW
NCH2 = ECH // NS
NEX = ECH - NCH * NW
SUP = 2 * CH
NSUPT = E // SUP
NSUP = NSUPT // NS
NEXS = NSUPT - NSUP * NS
RPT = NPAD // NS
TB = 2048
TG = NPAD // TB


@functools.partial(
    pl.kernel,
    out_type=[
        jax.ShapeDtypeStruct((ECH, CH), jnp.int32),
        jax.ShapeDtypeStruct((ECH, CH), jnp.int32),
        jax.ShapeDtypeStruct((NC * NPAD,), jnp.float32),
    ],
    mesh=plsc.VectorSubcoreMesh(core_axis_name="c", subcore_axis_name="s"),
    scratch_types=[
        pltpu.VMEM((NCH + 1, CH), jnp.int32),
        pltpu.VMEM((NCH + 1, CH), jnp.int32),
        pltpu.VMEM((NCH + 1, CH), jnp.int32),
        pltpu.VMEM((NCH + 1, CH), jnp.int32),
        pltpu.VMEM((NCH + 1, CH), jnp.float32),
        pltpu.VMEM((RPT,), jnp.float32),
        pltpu.VMEM_SHARED((NPAD,), jnp.float32),
        pltpu.SemaphoreType.DMA,
    ],
    compiler_params=pltpu.CompilerParams(use_tc_tiling_on_sc=False),
)
def _sc_deg(ei3, rml2d, rmh2d, deg_part,
            row_v, col_v, rml_v, rmh_v, ew_v, zb, deg_sh, ssem):
    cid = lax.axis_index("c")
    sid = lax.axis_index("s")
    wid = cid * NS + sid
    has_ex = wid >= NW - NEX
    exrow = NCH * NW + wid - (NW - NEX)

    def zbody(i, carry):
        zb[pl.ds(i * 16, 16)] = jnp.zeros((16,), jnp.float32)
        return carry

    lax.fori_loop(0, RPT // 16, zbody, 0)
    pltpu.sync_copy(zb, deg_sh.at[pl.ds(sid * RPT, RPT)])
    plsc.subcore_barrier()

    pltpu.sync_copy(ei3.at[0, pl.ds(wid * NCH, NCH)], row_v.at[pl.ds(0, NCH)])
    pltpu.sync_copy(ei3.at[1, pl.ds(wid * NCH, NCH)], col_v.at[pl.ds(0, NCH)])

    @pl.when(has_ex)
    def _ldex():
        pltpu.sync_copy(ei3.at[0, pl.ds(exrow, 1)], row_v.at[pl.ds(NCH, 1)])
        pltpu.sync_copy(ei3.at[1, pl.ds(exrow, 1)], col_v.at[pl.ds(NCH, 1)])

    dummy = jnp.int32(N + 1) + lax.iota(jnp.int32, 16)

    def _chunk(j):
        for k in range(8):
            sl = pl.ds(k * 16, 16)
            r = row_v[j, sl]
            c = col_v[j, sl]
            eq = r == c
            ew_v[j, sl] = jnp.where(eq, jnp.float32(0), jnp.float32(1))
            rm2 = jnp.where(eq, dummy, r)
            rm2 = rm2 + rm2
            rml_v[j, sl] = rm2
            rmh_v[j, sl] = rm2 + 1

    def ebody(j, carry):
        _chunk(j)
        pltpu.async_copy(ew_v.at[j], deg_sh.at[row_v.at[j]], ssem, add=True)
        return carry

    lax.fori_loop(0, NCH, ebody, 0)

    @pl.when(has_ex)
    def _exc():
        _chunk(NCH)
        pltpu.sync_copy(ew_v.at[NCH], deg_sh.at[row_v.at[NCH]], add=True)

    def dbody(j, carry):
        pltpu.make_async_copy(
            ew_v.at[j], deg_sh.at[row_v.at[j]], ssem).wait()
        return carry

    lax.fori_loop(0, NCH, dbody, 0)
    plsc.subcore_barrier()
    pltpu.sync_copy(rml_v.at[pl.ds(0, NCH)], rml2d.at[pl.ds(wid * NCH, NCH)])
    pltpu.sync_copy(rmh_v.at[pl.ds(0, NCH)], rmh2d.at[pl.ds(wid * NCH, NCH)])

    @pl.when(has_ex)
    def _stex():
        pltpu.sync_copy(rml_v.at[pl.ds(NCH, 1)], rml2d.at[pl.ds(exrow, 1)])
        pltpu.sync_copy(rmh_v.at[pl.ds(NCH, 1)], rmh2d.at[pl.ds(exrow, 1)])

    pltpu.sync_copy(deg_sh.at[pl.ds(sid * RPT, RPT)], zb)
    pltpu.sync_copy(zb, deg_part.at[pl.ds(cid * NPAD + sid * RPT, RPT)])


@functools.partial(
    pl.kernel,
    out_type=jax.ShapeDtypeStruct((NPAD, D), jnp.float32),
    mesh=plsc.VectorSubcoreMesh(core_axis_name="c", subcore_axis_name="s"),
    scratch_types=[
        pltpu.VMEM(((NSUP + 1) * SUP,), jnp.int32),
        pltpu.VMEM(((NSUP + 1) * SUP,), jnp.int32),
        pltpu.VMEM((2, SUP, HD), jnp.float32),
        pltpu.VMEM((CH, HD), jnp.float32),
        pltpu.VMEM_SHARED((NPAD, HD), jnp.float32),
        pltpu.SemaphoreType.DMA,
        pltpu.SemaphoreType.DMA,
        pltpu.SemaphoreType.DMA,
    ],
    compiler_params=pltpu.CompilerParams(use_tc_tiling_on_sc=False),
)
def _sc_msg(rml3, rmh3, col3, ycat, acc,
            rm_v, col_v, gbuf, zbuf, acc_sh, g0, g1, ssem):
    cid = lax.axis_index("c")
    sid = lax.axis_index("s")
    gsems = (g0, g1)
    has_ex = sid >= NS - NEXS
    exrow = NSUP * NS + sid - (NS - NEXS)

    def zbody(i, carry):
        for k in range(HD // 16):
            zbuf[i, pl.ds(k * 16, 16)] = jnp.zeros((16,), jnp.float32)
        return carry

    lax.fori_loop(0, CH, zbody, 0)
    base = sid * RPT
    for v in range(RPT // CH):
        pltpu.sync_copy(zbuf, acc_sh.at[pl.ds(base + v * CH, CH)])
    plsc.subcore_barrier()

    @pl.when(cid == 0)
    def _lo():
        pltpu.sync_copy(rml3.at[pl.ds(sid * NSUP * SUP, NSUP * SUP)],
                        rm_v.at[pl.ds(0, NSUP * SUP)])

        @pl.when(has_ex)
        def _lox():
            pltpu.sync_copy(rml3.at[pl.ds(exrow * SUP, SUP)],
                            rm_v.at[pl.ds(NSUP * SUP, SUP)])

    @pl.when(cid == 1)
    def _hi():
        pltpu.sync_copy(rmh3.at[pl.ds(sid * NSUP * SUP, NSUP * SUP)],
                        rm_v.at[pl.ds(0, NSUP * SUP)])

        @pl.when(has_ex)
        def _hix():
            pltpu.sync_copy(rmh3.at[pl.ds(exrow * SUP, SUP)],
                            rm_v.at[pl.ds(NSUP * SUP, SUP)])

    pltpu.sync_copy(col3.at[pl.ds(sid * NSUP * SUP, NSUP * SUP)],
                    col_v.at[pl.ds(0, NSUP * SUP)])

    @pl.when(has_ex)
    def _cx():
        pltpu.sync_copy(col3.at[pl.ds(exrow * SUP, SUP)],
                        col_v.at[pl.ds(NSUP * SUP, SUP)])

    NJO = NSUP // 2

    def gidx(js):
        return rm_v.at[pl.ds(js * SUP, SUP)]

    def sidx(js):
        return col_v.at[pl.ds(js * SUP, SUP)]

    pltpu.async_copy(ycat.at[gidx(0)], gbuf.at[0], gsems[0])

    def body(jo, carry):
        for t in range(2):
            js = 2 * jo + t
            fb = (t + 1) % 2
            if t == 0:
                @pl.when(jo > 0)
                def _ws():
                    pltpu.make_async_copy(
                        gbuf.at[fb], acc_sh.at[sidx(js - 1)], ssem).wait()

                pltpu.async_copy(ycat.at[gidx(js + 1)], gbuf.at[fb],
                                 gsems[fb])
            else:
                pltpu.make_async_copy(
                    gbuf.at[fb], acc_sh.at[sidx(js - 1)], ssem).wait()

                @pl.when(jo < NJO - 1)
                def _fg():
                    pltpu.async_copy(ycat.at[gidx(js + 1)], gbuf.at[fb],
                                     gsems[fb])

            pltpu.make_async_copy(ycat.at[gidx(js)], gbuf.at[t],
                                  gsems[t]).wait()
            pltpu.async_copy(gbuf.at[t], acc_sh.at[sidx(js)], ssem,
                             add=True)
        return carry

    lax.fori_loop(0, NJO, body, 0)
    pltpu.make_async_copy(
        gbuf.at[1], acc_sh.at[sidx(NSUP - 1)], ssem).wait()

    @pl.when(has_ex)
    def _exc():
        pltpu.sync_copy(ycat.at[gidx(NSUP)], gbuf.at[0])
        pltpu.sync_copy(gbuf.at[0], acc_sh.at[sidx(NSUP)], add=True)

    plsc.subcore_barrier()
    for v in range(RPT // CH):
        pltpu.sync_copy(acc_sh.at[pl.ds(base + v * CH, CH)], zbuf)
        pltpu.sync_copy(zbuf, acc.at[pl.ds(base + v * CH, CH),
                                     pl.ds(cid * HD, HD)])


def _disc(dp_ref):
    parts = []
    r = lax.broadcasted_iota(jnp.int32, (128, 128), 0)
    c = lax.broadcasted_iota(jnp.int32, (128, 128), 1)
    eye = r == c
    for g in range(TB // 128):
        sl = slice(g * 128, (g + 1) * 128)
        deg = dp_ref[0:1, sl] + dp_ref[1:2, sl] + 1.0
        disl = lax.rsqrt(deg)
        dmat = jnp.where(eye, jnp.broadcast_to(disl, (128, 128)), 0.0)
        parts.append(jnp.sum(dmat, axis=1, keepdims=True))
    return jnp.concatenate(parts, axis=0)


def _tc_prep(x_ref, w_ref, dp_ref, y_ref):
    i = pl.program_id(0)
    gr = i * TB + lax.broadcasted_iota(jnp.int32, (TB, 1), 0)
    y = _disc(dp_ref) * (x_ref[...] * w_ref[0:1, :])
    y_ref[...] = jnp.where(gr < N, y, 0.0)


def _tc_comb(acc_ref, y_ref, dp_ref, b_ref, out_ref):
    s = acc_ref[...] + y_ref[...]
    out_ref[...] = _disc(dp_ref) * s + b_ref[0:1, :]


def kernel(x, edge_index, W, b):
    ei3 = edge_index.reshape(2, ECH, CH)

    rml2d, rmh2d, deg_part = _sc_deg(ei3)

    wb = jnp.broadcast_to(W, (8, D))
    dp8 = jnp.pad(deg_part.reshape(NC, NPAD), ((0, 6), (0, 0)))
    y = pl.pallas_call(
        _tc_prep,
        grid=(TG,),
        in_specs=[
            pl.BlockSpec((TB, D), lambda i: (i, 0)),
            pl.BlockSpec((8, D), lambda i: (0, 0)),
            pl.BlockSpec((8, TB), lambda i: (0, i)),
        ],
        out_specs=pl.BlockSpec((TB, D), lambda i: (i, 0)),
        out_shape=jax.ShapeDtypeStruct((NPAD, D), jnp.float32),
    )(x, wb, dp8)

    rml3 = rml2d.reshape(E)
    rmh3 = rmh2d.reshape(E)
    col3 = ei3[1].reshape(E)
    acc = _sc_msg(rml3, rmh3, col3, y.reshape(2 * NPAD, HD))

    bb = jnp.broadcast_to(b[None, :], (8, D))
    out = pl.pallas_call(
        _tc_comb,
        grid=(TG,),
        in_specs=[
            pl.BlockSpec((TB, D), lambda i: (i, 0)),
            pl.BlockSpec((TB, D), lambda i: (i, 0)),
            pl.BlockSpec((8, TB), lambda i: (0, i)),
            pl.BlockSpec((8, D), lambda i: (0, 0)),
        ],
        out_specs=pl.BlockSpec((TB, D), lambda i: (i, 0)),
        out_shape=jax.ShapeDtypeStruct((N, D), jnp.float32),
    )(acc, y, dp8, bb)

    return out


# speedup vs baseline: 1.0715x; 1.0715x over previous
import functools

import jax
import jax.numpy as jnp
from jax import lax
from jax.experimental import pallas as pl
from jax.experimental.pallas import tpu as pltpu
from jax.experimental.pallas import tpu_sc as plsc

N = 10000
D = 128
HD = D // 2
E = 320000
NPAD = 10240
NC, NS = 2, 16
NW = NC * NS
CH = 128
ECH = E // CH
NCH = ECH // NW
NCH2 = ECH // NS
NEX = ECH - NCH * NW
SUP = CH
NSUPT = E // SUP
NSUP = NSUPT // NS
NEXS = NSUPT - NSUP * NS
RPT = NPAD // NS
TB = 2048
TG = NPAD // TB


@functools.partial(
    pl.kernel,
    out_type=[
        jax.ShapeDtypeStruct((ECH, CH), jnp.int32),
        jax.ShapeDtypeStruct((ECH, CH), jnp.int32),
        jax.ShapeDtypeStruct((NC * NPAD,), jnp.float32),
    ],
    mesh=plsc.VectorSubcoreMesh(core_axis_name="c", subcore_axis_name="s"),
    scratch_types=[
        pltpu.VMEM((NCH + 1, CH), jnp.int32),
        pltpu.VMEM((NCH + 1, CH), jnp.int32),
        pltpu.VMEM((NCH + 1, CH), jnp.int32),
        pltpu.VMEM((NCH + 1, CH), jnp.int32),
        pltpu.VMEM((NCH + 1, CH), jnp.float32),
        pltpu.VMEM((RPT,), jnp.float32),
        pltpu.VMEM_SHARED((NPAD,), jnp.float32),
        pltpu.SemaphoreType.DMA,
    ],
    compiler_params=pltpu.CompilerParams(use_tc_tiling_on_sc=False),
)
def _sc_deg(ei3, rml2d, rmh2d, deg_part,
            row_v, col_v, rml_v, rmh_v, ew_v, zb, deg_sh, ssem):
    cid = lax.axis_index("c")
    sid = lax.axis_index("s")
    wid = cid * NS + sid
    has_ex = wid >= NW - NEX
    exrow = NCH * NW + wid - (NW - NEX)

    def zbody(i, carry):
        zb[pl.ds(i * 16, 16)] = jnp.zeros((16,), jnp.float32)
        return carry

    lax.fori_loop(0, RPT // 16, zbody, 0)
    pltpu.sync_copy(zb, deg_sh.at[pl.ds(sid * RPT, RPT)])
    plsc.subcore_barrier()

    pltpu.sync_copy(ei3.at[0, pl.ds(wid * NCH, NCH)], row_v.at[pl.ds(0, NCH)])
    pltpu.sync_copy(ei3.at[1, pl.ds(wid * NCH, NCH)], col_v.at[pl.ds(0, NCH)])

    @pl.when(has_ex)
    def _ldex():
        pltpu.sync_copy(ei3.at[0, pl.ds(exrow, 1)], row_v.at[pl.ds(NCH, 1)])
        pltpu.sync_copy(ei3.at[1, pl.ds(exrow, 1)], col_v.at[pl.ds(NCH, 1)])

    dummy = jnp.int32(N + 1) + lax.iota(jnp.int32, 16)

    def _chunk(j):
        for k in range(8):
            sl = pl.ds(k * 16, 16)
            r = row_v[j, sl]
            c = col_v[j, sl]
            eq = r == c
            ew_v[j, sl] = jnp.where(eq, jnp.float32(0), jnp.float32(1))
            rm2 = jnp.where(eq, dummy, r)
            rm2 = rm2 + rm2
            rml_v[j, sl] = rm2
            rmh_v[j, sl] = rm2 + 1

    def ebody(j, carry):
        _chunk(j)
        pltpu.async_copy(ew_v.at[j], deg_sh.at[row_v.at[j]], ssem, add=True)
        return carry

    lax.fori_loop(0, NCH, ebody, 0)

    @pl.when(has_ex)
    def _exc():
        _chunk(NCH)
        pltpu.sync_copy(ew_v.at[NCH], deg_sh.at[row_v.at[NCH]], add=True)

    def dbody(j, carry):
        pltpu.make_async_copy(
            ew_v.at[j], deg_sh.at[row_v.at[j]], ssem).wait()
        return carry

    lax.fori_loop(0, NCH, dbody, 0)
    plsc.subcore_barrier()
    pltpu.sync_copy(rml_v.at[pl.ds(0, NCH)], rml2d.at[pl.ds(wid * NCH, NCH)])
    pltpu.sync_copy(rmh_v.at[pl.ds(0, NCH)], rmh2d.at[pl.ds(wid * NCH, NCH)])

    @pl.when(has_ex)
    def _stex():
        pltpu.sync_copy(rml_v.at[pl.ds(NCH, 1)], rml2d.at[pl.ds(exrow, 1)])
        pltpu.sync_copy(rmh_v.at[pl.ds(NCH, 1)], rmh2d.at[pl.ds(exrow, 1)])

    pltpu.sync_copy(deg_sh.at[pl.ds(sid * RPT, RPT)], zb)
    pltpu.sync_copy(zb, deg_part.at[pl.ds(cid * NPAD + sid * RPT, RPT)])


@functools.partial(
    pl.kernel,
    out_type=jax.ShapeDtypeStruct((NPAD, D), jnp.float32),
    mesh=plsc.VectorSubcoreMesh(core_axis_name="c", subcore_axis_name="s"),
    scratch_types=[
        pltpu.VMEM(((NSUP + 1) * SUP,), jnp.int32),
        pltpu.VMEM(((NSUP + 1) * SUP,), jnp.int32),
        pltpu.VMEM((4, SUP, HD), jnp.float32),
        pltpu.VMEM((CH, HD), jnp.float32),
        pltpu.VMEM_SHARED((NPAD, HD), jnp.float32),
        pltpu.SemaphoreType.DMA,
        pltpu.SemaphoreType.DMA,
        pltpu.SemaphoreType.DMA,
        pltpu.SemaphoreType.DMA,
        pltpu.SemaphoreType.DMA,
    ],
    compiler_params=pltpu.CompilerParams(use_tc_tiling_on_sc=False),
)
def _sc_msg(rml3, rmh3, col3, ycat, acc,
            rm_v, col_v, gbuf, zbuf, acc_sh, g0, g1, g2, g3, ssem):
    cid = lax.axis_index("c")
    sid = lax.axis_index("s")
    gsems = (g0, g1, g2, g3)
    has_ex = sid >= NS - NEXS
    exrow = NSUP * NS + sid - (NS - NEXS)

    def zbody(i, carry):
        for k in range(HD // 16):
            zbuf[i, pl.ds(k * 16, 16)] = jnp.zeros((16,), jnp.float32)
        return carry

    lax.fori_loop(0, CH, zbody, 0)
    base = sid * RPT
    for v in range(RPT // CH):
        pltpu.sync_copy(zbuf, acc_sh.at[pl.ds(base + v * CH, CH)])
    plsc.subcore_barrier()

    @pl.when(cid == 0)
    def _lo():
        pltpu.sync_copy(rml3.at[pl.ds(sid * NSUP * SUP, NSUP * SUP)],
                        rm_v.at[pl.ds(0, NSUP * SUP)])

        @pl.when(has_ex)
        def _lox():
            pltpu.sync_copy(rml3.at[pl.ds(exrow * SUP, SUP)],
                            rm_v.at[pl.ds(NSUP * SUP, SUP)])

    @pl.when(cid == 1)
    def _hi():
        pltpu.sync_copy(rmh3.at[pl.ds(sid * NSUP * SUP, NSUP * SUP)],
                        rm_v.at[pl.ds(0, NSUP * SUP)])

        @pl.when(has_ex)
        def _hix():
            pltpu.sync_copy(rmh3.at[pl.ds(exrow * SUP, SUP)],
                            rm_v.at[pl.ds(NSUP * SUP, SUP)])

    pltpu.sync_copy(col3.at[pl.ds(sid * NSUP * SUP, NSUP * SUP)],
                    col_v.at[pl.ds(0, NSUP * SUP)])

    @pl.when(has_ex)
    def _cx():
        pltpu.sync_copy(col3.at[pl.ds(exrow * SUP, SUP)],
                        col_v.at[pl.ds(NSUP * SUP, SUP)])

    NJO = NSUP // 4

    def gidx(js):
        return rm_v.at[pl.ds(js * SUP, SUP)]

    def sidx(js):
        return col_v.at[pl.ds(js * SUP, SUP)]

    for q in range(3):
        pltpu.async_copy(ycat.at[gidx(q)], gbuf.at[q], gsems[q])

    def body(jo, carry):
        for t in range(4):
            js = 4 * jo + t
            fb = (t + 3) % 4
            if t == 0:
                @pl.when(jo > 0)
                def _ws():
                    pltpu.make_async_copy(
                        gbuf.at[fb], acc_sh.at[sidx(js - 1)], ssem).wait()

                pltpu.async_copy(ycat.at[gidx(js + 3)], gbuf.at[fb],
                                 gsems[fb])
            else:
                pltpu.make_async_copy(
                    gbuf.at[fb], acc_sh.at[sidx(js - 1)], ssem).wait()

                @pl.when(jo < NJO - 1)
                def _fg():
                    pltpu.async_copy(ycat.at[gidx(js + 3)], gbuf.at[fb],
                                     gsems[fb])

            pltpu.make_async_copy(ycat.at[gidx(js)], gbuf.at[t],
                                  gsems[t]).wait()
            pltpu.async_copy(gbuf.at[t], acc_sh.at[sidx(js)], ssem,
                             add=True)
        return carry

    lax.fori_loop(0, NJO, body, 0)
    pltpu.make_async_copy(
        gbuf.at[3], acc_sh.at[sidx(NSUP - 1)], ssem).wait()

    @pl.when(has_ex)
    def _exc():
        pltpu.sync_copy(ycat.at[gidx(NSUP)], gbuf.at[0])
        pltpu.sync_copy(gbuf.at[0], acc_sh.at[sidx(NSUP)], add=True)

    plsc.subcore_barrier()
    for v in range(RPT // CH):
        pltpu.sync_copy(acc_sh.at[pl.ds(base + v * CH, CH)], zbuf)
        pltpu.sync_copy(zbuf, acc.at[pl.ds(base + v * CH, CH),
                                     pl.ds(cid * HD, HD)])


def _disc(dp_ref):
    parts = []
    r = lax.broadcasted_iota(jnp.int32, (128, 128), 0)
    c = lax.broadcasted_iota(jnp.int32, (128, 128), 1)
    eye = r == c
    for g in range(TB // 128):
        sl = slice(g * 128, (g + 1) * 128)
        deg = dp_ref[0:1, sl] + dp_ref[1:2, sl] + 1.0
        disl = lax.rsqrt(deg)
        dmat = jnp.where(eye, jnp.broadcast_to(disl, (128, 128)), 0.0)
        parts.append(jnp.sum(dmat, axis=1, keepdims=True))
    return jnp.concatenate(parts, axis=0)


def _tc_prep(x_ref, w_ref, dp_ref, y_ref):
    i = pl.program_id(0)
    gr = i * TB + lax.broadcasted_iota(jnp.int32, (TB, 1), 0)
    y = _disc(dp_ref) * (x_ref[...] * w_ref[0:1, :])
    y_ref[...] = jnp.where(gr < N, y, 0.0)


def _tc_comb(acc_ref, y_ref, dp_ref, b_ref, out_ref):
    s = acc_ref[...] + y_ref[...]
    out_ref[...] = _disc(dp_ref) * s + b_ref[0:1, :]


def kernel(x, edge_index, W, b):
    ei3 = edge_index.reshape(2, ECH, CH)

    rml2d, rmh2d, deg_part = _sc_deg(ei3)

    wb = jnp.broadcast_to(W, (8, D))
    dp8 = jnp.pad(deg_part.reshape(NC, NPAD), ((0, 6), (0, 0)))
    y = pl.pallas_call(
        _tc_prep,
        grid=(TG,),
        in_specs=[
            pl.BlockSpec((TB, D), lambda i: (i, 0)),
            pl.BlockSpec((8, D), lambda i: (0, 0)),
            pl.BlockSpec((8, TB), lambda i: (0, i)),
        ],
        out_specs=pl.BlockSpec((TB, D), lambda i: (i, 0)),
        out_shape=jax.ShapeDtypeStruct((NPAD, D), jnp.float32),
    )(x, wb, dp8)

    rml3 = rml2d.reshape(E)
    rmh3 = rmh2d.reshape(E)
    col3 = ei3[1].reshape(E)
    acc = _sc_msg(rml3, rmh3, col3, y.reshape(2 * NPAD, HD))

    bb = jnp.broadcast_to(b[None, :], (8, D))
    out = pl.pallas_call(
        _tc_comb,
        grid=(TG,),
        in_specs=[
            pl.BlockSpec((TB, D), lambda i: (i, 0)),
            pl.BlockSpec((TB, D), lambda i: (i, 0)),
            pl.BlockSpec((8, TB), lambda i: (0, i)),
            pl.BlockSpec((8, D), lambda i: (0, 0)),
        ],
        out_specs=pl.BlockSpec((TB, D), lambda i: (i, 0)),
        out_shape=jax.ShapeDtypeStruct((N, D), jnp.float32),
    )(acc, y, dp8, bb)

    return out
